# baseline (speedup 1.0000x reference)
import functools

import jax
import jax.numpy as jnp
from jax import lax
from jax.experimental import pallas as pl
from jax.experimental.pallas import tpu as pltpu
from jax.experimental.pallas import tpu_sc as plsc

NUM_NODES = 1000000
DIM = 64
BATCH = 16384
NUM_NEG = 20
MARGIN = 1.0

NC = 2
NS = 16
NW = NC * NS
LANES = 16

_COMPILER_PARAMS = pltpu.CompilerParams(needs_layout_passes=False,
                                        use_tc_tiling_on_sc=True)


NBLK = NUM_NODES // 128
MAIN_ITERS = NBLK // NW
EXTRA_BLKS = NBLK - MAIN_ITERS * NW
TAIL_NODES = NUM_NODES - NBLK * 128
BLK_WORDS = 128 * DIM
NSLOT = 4


def _transpose_body(embt_hbm, tail_hbm, out_hbm, in_v, out_v, isem, osem):
    wid = lax.axis_index("c") * NS + lax.axis_index("s")
    lane_iota = lax.iota(jnp.int32, LANES)

    def fire_in(b, s):
        pltpu.make_async_copy(
            embt_hbm.at[pl.ds(0, DIM), pl.ds(b * 128, 128)],
            in_v.at[s], isem.at[s]).start()

    def wait_in(s):
        pltpu.make_async_copy(
            embt_hbm.at[pl.ds(0, DIM), pl.ds(0, 128)],
            in_v.at[s], isem.at[s]).wait()

    def fire_out(b, s):
        pltpu.make_async_copy(
            out_v.at[pl.ds(s * BLK_WORDS, BLK_WORDS)],
            out_hbm.at[pl.ds(b * BLK_WORDS, BLK_WORDS)], osem.at[s]).start()

    def wait_out(s):
        pltpu.make_async_copy(
            out_v.at[pl.ds(0, BLK_WORDS)],
            out_hbm.at[pl.ds(0, BLK_WORDS)], osem.at[s]).wait()

    def transpose_block(s):
        src = in_v.at[s]
        obase = s * BLK_WORDS

        def j_body(j, _):
            for dd in range(4):
                dvec = jnp.full((LANES,), j * 4 + dd, jnp.int32)
                for nsub in range(8):
                    nvec = lane_iota + nsub * LANES
                    val = plsc.load_gather(src, [dvec, nvec])
                    plsc.store_scatter(out_v, [obase + nvec * DIM + dvec], val)
            return 0

        lax.fori_loop(0, DIM // 4, j_body, 0)

    for p in range(NSLOT - 1):
        fire_in(p * NW + wid, p)

    def main_body(i, _):
        s = lax.rem(i, NSLOT)

        @pl.when(i + NSLOT - 1 < MAIN_ITERS)
        def _():
            fire_in((i + NSLOT - 1) * NW + wid, lax.rem(i + NSLOT - 1, NSLOT))

        wait_in(s)

        @pl.when(i >= NSLOT)
        def _():
            wait_out(s)

        transpose_block(s)
        fire_out(i * NW + wid, s)
        return 0

    lax.fori_loop(0, MAIN_ITERS, main_body, 0)
    for s in range(NSLOT):
        wait_out(s)

    @pl.when(wid < EXTRA_BLKS)
    def _():
        b = MAIN_ITERS * NW + wid
        pltpu.sync_copy(embt_hbm.at[pl.ds(0, DIM), pl.ds(b * 128, 128)],
                        in_v.at[0])
        transpose_block(0)
        pltpu.sync_copy(out_v.at[pl.ds(0, BLK_WORDS)],
                        out_hbm.at[pl.ds(b * BLK_WORDS, BLK_WORDS)])

    @pl.when(wid == EXTRA_BLKS)
    def _():
        pltpu.sync_copy(tail_hbm, out_v.at[pl.ds(0, TAIL_NODES * DIM)])
        pltpu.sync_copy(out_v.at[pl.ds(0, TAIL_NODES * DIM)],
                        out_hbm.at[pl.ds(NBLK * BLK_WORDS, TAIL_NODES * DIM)])


B_PER_W = BATCH // NW
CHUNK = 32
N_CHUNKS = B_PER_W // CHUNK
NEG_PER_CHUNK = CHUNK * NUM_NEG
NEG_IDX_ROWS = NEG_PER_CHUNK // 128
NEG_IDX_ROWS_W = B_PER_W * NUM_NEG // 128
GROUPS = CHUNK // LANES
UNROLL = 4
NBUF = 2


def _loss_body(emb_hbm, tgt_hbm, ctx_hbm, neg_hbm, out_hbm,
             t_idx, c_idx, n_idx, t_rows, c_rows, n_rows, acc_v, sem):
    wid = lax.axis_index("c") * NS + lax.axis_index("s")
    lane_iota = lax.iota(jnp.int32, LANES)
    zeros = jnp.zeros((LANES,), jnp.float32)

    pltpu.sync_copy(tgt_hbm.at[pl.ds(wid * B_PER_W, B_PER_W)], t_idx)
    pltpu.sync_copy(ctx_hbm.at[pl.ds(wid * B_PER_W, B_PER_W)], c_idx)
    pltpu.sync_copy(neg_hbm.at[pl.ds(wid * NEG_IDX_ROWS_W, NEG_IDX_ROWS_W)],
                    n_idx)

    def fire(ci, buf):
        cps = [pltpu.make_async_copy(
                   emb_hbm.at[t_idx.at[pl.ds(ci * CHUNK, CHUNK)]],
                   t_rows.at[buf], sem.at[buf]),
               pltpu.make_async_copy(
                   emb_hbm.at[c_idx.at[pl.ds(ci * CHUNK, CHUNK)]],
                   c_rows.at[buf], sem.at[buf])]
        cps += [pltpu.make_async_copy(emb_hbm.at[n_idx.at[ci * NEG_IDX_ROWS + j]],
                                      n_rows.at[buf].at[pl.ds(j * 128, 128)],
                                      sem.at[buf])
                for j in range(NEG_IDX_ROWS)]
        for cp in cps:
            cp.start()
        return cps

    def wait(ci, buf):
        for cp in fire_descr(buf):
            cp.wait()

    def fire_descr(buf):
        cps = [pltpu.make_async_copy(
                   emb_hbm.at[t_idx.at[pl.ds(0, CHUNK)]], t_rows.at[buf],
                   sem.at[buf]),
               pltpu.make_async_copy(
                   emb_hbm.at[c_idx.at[pl.ds(0, CHUNK)]], c_rows.at[buf],
                   sem.at[buf])]
        cps += [pltpu.make_async_copy(emb_hbm.at[n_idx.at[j]],
                                      n_rows.at[buf].at[pl.ds(j * 128, 128)],
                                      sem.at[buf])
                for j in range(NEG_IDX_ROWS)]
        return cps

    def compute(buf, loss_acc):
        t_r, c_r, n_r = t_rows.at[buf], c_rows.at[buf], n_rows.at[buf]

        def group_body(g, acc):
            e_lanes = lane_iota + g * LANES
            n_base = e_lanes * NUM_NEG

            def d_body(j, carry):
                pos, negs = carry
                for dd in range(UNROLL):
                    dcol = (lane_iota + (j * UNROLL + dd)) & (DIM - 1)
                    t_d = plsc.load_gather(t_r, [e_lanes, dcol])
                    c_d = plsc.load_gather(c_r, [e_lanes, dcol])
                    pos = pos + t_d * c_d
                    negs = tuple(
                        negs[k]
                        + t_d * plsc.load_gather(n_r, [n_base + k, dcol])
                        for k in range(NUM_NEG))
                return pos, negs

            pos, negs = lax.fori_loop(
                0, DIM // UNROLL, d_body, (zeros, (zeros,) * NUM_NEG))
            contrib = zeros
            for k in range(NUM_NEG):
                contrib = contrib + jnp.maximum(MARGIN - pos + negs[k], 0.0)
            return acc + contrib

        return lax.fori_loop(0, GROUPS, group_body, loss_acc)

    fire(0, 0)

    def chunk_body(ci, loss_acc):
        buf = lax.rem(ci, NBUF)
        nbuf = lax.rem(ci + 1, NBUF)

        @pl.when(ci + 1 < N_CHUNKS)
        def _():
            fire(ci + 1, nbuf)

        wait(ci, buf)
        return compute(buf, loss_acc)

    loss = lax.fori_loop(0, N_CHUNKS, chunk_body, zeros)
    acc_v[...] = loss
    pltpu.sync_copy(acc_v, out_hbm.at[wid])


@jax.jit
def _margin_loss(embeddings, targets, contexts, negatives2d):
    mesh = plsc.VectorSubcoreMesh(core_axis_name="c", subcore_axis_name="s",
                                  num_cores=NC, num_subcores=NS)

    embt = embeddings.T
    tail = embeddings[NBLK * 128:].reshape(-1)

    table_flat = pl.kernel(
        _transpose_body,
        out_type=jax.ShapeDtypeStruct((NUM_NODES * DIM,), jnp.float32),
        mesh=mesh,
        scratch_types=[
            pltpu.VMEM((NSLOT, DIM, 128), jnp.float32),
            pltpu.VMEM((NSLOT * BLK_WORDS,), jnp.float32),
            pltpu.SemaphoreType.DMA((NSLOT,)),
            pltpu.SemaphoreType.DMA((NSLOT,)),
        ],
        compiler_params=pltpu.CompilerParams(needs_layout_passes=False,
                                             use_tc_tiling_on_sc=True),
    )(embt, tail)

    partials = pl.kernel(
        _loss_body,
        out_type=jax.ShapeDtypeStruct((NW, LANES), jnp.float32),
        mesh=mesh,
        scratch_types=[
            pltpu.VMEM((B_PER_W,), jnp.int32),
            pltpu.VMEM((B_PER_W,), jnp.int32),
            pltpu.VMEM((NEG_IDX_ROWS_W, 128), jnp.int32),
            pltpu.VMEM((NBUF, CHUNK, DIM), jnp.float32),
            pltpu.VMEM((NBUF, CHUNK, DIM), jnp.float32),
            pltpu.VMEM((NBUF, NEG_PER_CHUNK, DIM), jnp.float32),
            pltpu.VMEM((LANES,), jnp.float32),
            pltpu.SemaphoreType.DMA((NBUF,)),
        ],
        compiler_params=pltpu.CompilerParams(needs_layout_passes=False,
                                             use_tc_tiling_on_sc=False),
    )(table_flat.reshape(NUM_NODES, DIM), targets, contexts, negatives2d)
    return jnp.sum(partials) / (BATCH * NUM_NEG)


def kernel(embeddings, targets, contexts, negatives):
    targets = targets.astype(jnp.int32)
    contexts = contexts.astype(jnp.int32)
    negatives2d = negatives.astype(jnp.int32).reshape(-1, 128)
    return _margin_loss(embeddings, targets, contexts, negatives2d)


# speedup vs baseline: 2.0724x; 2.0724x over previous
import functools

import jax
import jax.numpy as jnp
from jax import lax
from jax.experimental import pallas as pl
from jax.experimental.pallas import tpu as pltpu
from jax.experimental.pallas import tpu_sc as plsc

NUM_NODES = 1000000
DIM = 64
BATCH = 16384
NUM_NEG = 20
MARGIN = 1.0

NC = 2
NS = 16
NW = NC * NS
LANES = 16

_COMPILER_PARAMS = pltpu.CompilerParams(needs_layout_passes=False,
                                        use_tc_tiling_on_sc=True)


NBLK = NUM_NODES // 128
MAIN_ITERS = NBLK // NW
EXTRA_BLKS = NBLK - MAIN_ITERS * NW
TAIL_NODES = NUM_NODES - NBLK * 128
BLK_WORDS = 128 * DIM
NSLOT = 4


def _transpose_body(embt_hbm, tail_hbm, out_hbm, in_v, out_v, isem, osem):
    wid = lax.axis_index("c") * NS + lax.axis_index("s")
    lane_iota = lax.iota(jnp.int32, LANES)

    def fire_in(b, s):
        for dh in range(8):
            pltpu.make_async_copy(
                embt_hbm.at[pl.ds(dh * 8, 8), pl.ds(b * 128, 128)],
                in_v.at[s].at[dh], isem.at[s]).start()

    def wait_in(s):
        for dh in range(8):
            pltpu.make_async_copy(
                embt_hbm.at[pl.ds(dh * 8, 8), pl.ds(0, 128)],
                in_v.at[s].at[dh], isem.at[s]).wait()

    def fire_out(b, s):
        pltpu.make_async_copy(
            out_v.at[pl.ds(s * BLK_WORDS, BLK_WORDS)],
            out_hbm.at[pl.ds(b * BLK_WORDS, BLK_WORDS)], osem.at[s]).start()

    def wait_out(s):
        pltpu.make_async_copy(
            out_v.at[pl.ds(0, BLK_WORDS)],
            out_hbm.at[pl.ds(0, BLK_WORDS)], osem.at[s]).wait()

    def transpose_block(s):
        src = in_v.at[s]
        obase = s * BLK_WORDS
        nvecs = [lane_iota + nsub * LANES for nsub in range(8)]
        obases = [obase + nv * DIM for nv in nvecs]

        def j_body(j, _):
            for dd in range(4):
                dvec = (lane_iota + (j * 4 + dd)) & (DIM - 1)
                dh = dvec >> 3
                dl = dvec & 7
                for nsub in range(8):
                    val = plsc.load_gather(src, [dh, dl, nvecs[nsub]])
                    plsc.store_scatter(out_v, [obases[nsub] + dvec], val)
            return 0

        lax.fori_loop(0, DIM // 4, j_body, 0)

    for p in range(NSLOT - 1):
        fire_in(p * NW + wid, p)

    def main_body(i, _):
        s = lax.rem(i, NSLOT)

        @pl.when(i + NSLOT - 1 < MAIN_ITERS)
        def _():
            fire_in((i + NSLOT - 1) * NW + wid, lax.rem(i + NSLOT - 1, NSLOT))

        wait_in(s)

        @pl.when(i >= NSLOT)
        def _():
            wait_out(s)

        transpose_block(s)
        fire_out(i * NW + wid, s)
        return 0

    lax.fori_loop(0, MAIN_ITERS, main_body, 0)
    for s in range(NSLOT):
        wait_out(s)

    @pl.when(wid < EXTRA_BLKS)
    def _():
        b = MAIN_ITERS * NW + wid
        for dh in range(8):
            pltpu.sync_copy(embt_hbm.at[pl.ds(dh * 8, 8), pl.ds(b * 128, 128)],
                            in_v.at[0].at[dh])
        transpose_block(0)
        pltpu.sync_copy(out_v.at[pl.ds(0, BLK_WORDS)],
                        out_hbm.at[pl.ds(b * BLK_WORDS, BLK_WORDS)])

    @pl.when(wid == EXTRA_BLKS)
    def _():
        pltpu.sync_copy(tail_hbm, out_v.at[pl.ds(0, TAIL_NODES * DIM)])
        pltpu.sync_copy(out_v.at[pl.ds(0, TAIL_NODES * DIM)],
                        out_hbm.at[pl.ds(NBLK * BLK_WORDS, TAIL_NODES * DIM)])


B_PER_W = BATCH // NW
CHUNK = 32
N_CHUNKS = B_PER_W // CHUNK
NEG_PER_CHUNK = CHUNK * NUM_NEG
NEG_IDX_ROWS = NEG_PER_CHUNK // 128
NEG_IDX_ROWS_W = B_PER_W * NUM_NEG // 128
GROUPS = CHUNK // LANES
UNROLL = 4
NBUF = 2


def _loss_body(emb_hbm, tgt_hbm, ctx_hbm, neg_hbm, out_hbm,
             t_idx, c_idx, n_idx, t_rows, c_rows, n_rows, acc_v, sem):
    wid = lax.axis_index("c") * NS + lax.axis_index("s")
    lane_iota = lax.iota(jnp.int32, LANES)
    zeros = jnp.zeros((LANES,), jnp.float32)

    pltpu.sync_copy(tgt_hbm.at[pl.ds(wid * B_PER_W, B_PER_W)], t_idx)
    pltpu.sync_copy(ctx_hbm.at[pl.ds(wid * B_PER_W, B_PER_W)], c_idx)
    pltpu.sync_copy(neg_hbm.at[pl.ds(wid * NEG_IDX_ROWS_W, NEG_IDX_ROWS_W)],
                    n_idx)

    def fire(ci, buf):
        cps = [pltpu.make_async_copy(
                   emb_hbm.at[t_idx.at[pl.ds(ci * CHUNK, CHUNK)]],
                   t_rows.at[buf], sem.at[buf]),
               pltpu.make_async_copy(
                   emb_hbm.at[c_idx.at[pl.ds(ci * CHUNK, CHUNK)]],
                   c_rows.at[buf], sem.at[buf])]
        cps += [pltpu.make_async_copy(emb_hbm.at[n_idx.at[ci * NEG_IDX_ROWS + j]],
                                      n_rows.at[buf].at[pl.ds(j * 128, 128)],
                                      sem.at[buf])
                for j in range(NEG_IDX_ROWS)]
        for cp in cps:
            cp.start()
        return cps

    def wait(ci, buf):
        for cp in fire_descr(buf):
            cp.wait()

    def fire_descr(buf):
        cps = [pltpu.make_async_copy(
                   emb_hbm.at[t_idx.at[pl.ds(0, CHUNK)]], t_rows.at[buf],
                   sem.at[buf]),
               pltpu.make_async_copy(
                   emb_hbm.at[c_idx.at[pl.ds(0, CHUNK)]], c_rows.at[buf],
                   sem.at[buf])]
        cps += [pltpu.make_async_copy(emb_hbm.at[n_idx.at[j]],
                                      n_rows.at[buf].at[pl.ds(j * 128, 128)],
                                      sem.at[buf])
                for j in range(NEG_IDX_ROWS)]
        return cps

    def compute(buf, loss_acc):
        t_r, c_r, n_r = t_rows.at[buf], c_rows.at[buf], n_rows.at[buf]

        def group_body(g, acc):
            e_lanes = lane_iota + g * LANES
            n_base = e_lanes * NUM_NEG

            def d_body(j, carry):
                pos, negs = carry
                for dd in range(UNROLL):
                    dcol = (lane_iota + (j * UNROLL + dd)) & (DIM - 1)
                    t_d = plsc.load_gather(t_r, [e_lanes, dcol])
                    c_d = plsc.load_gather(c_r, [e_lanes, dcol])
                    pos = pos + t_d * c_d
                    negs = tuple(
                        negs[k]
                        + t_d * plsc.load_gather(n_r, [n_base + k, dcol])
                        for k in range(NUM_NEG))
                return pos, negs

            pos, negs = lax.fori_loop(
                0, DIM // UNROLL, d_body, (zeros, (zeros,) * NUM_NEG))
            contrib = zeros
            for k in range(NUM_NEG):
                contrib = contrib + jnp.maximum(MARGIN - pos + negs[k], 0.0)
            return acc + contrib

        return lax.fori_loop(0, GROUPS, group_body, loss_acc)

    fire(0, 0)

    def chunk_body(ci, loss_acc):
        buf = lax.rem(ci, NBUF)
        nbuf = lax.rem(ci + 1, NBUF)

        @pl.when(ci + 1 < N_CHUNKS)
        def _():
            fire(ci + 1, nbuf)

        wait(ci, buf)
        return compute(buf, loss_acc)

    loss = lax.fori_loop(0, N_CHUNKS, chunk_body, zeros)
    acc_v[...] = loss
    pltpu.sync_copy(acc_v, out_hbm.at[wid])


@jax.jit
def _margin_loss(embeddings, targets, contexts, negatives2d):
    mesh = plsc.VectorSubcoreMesh(core_axis_name="c", subcore_axis_name="s",
                                  num_cores=NC, num_subcores=NS)

    embt = embeddings.T
    tail = embeddings[NBLK * 128:].reshape(-1)

    table_flat = pl.kernel(
        _transpose_body,
        out_type=jax.ShapeDtypeStruct((NUM_NODES * DIM,), jnp.float32),
        mesh=mesh,
        scratch_types=[
            pltpu.VMEM((NSLOT, 8, 8, 128), jnp.float32),
            pltpu.VMEM((NSLOT * BLK_WORDS,), jnp.float32),
            pltpu.SemaphoreType.DMA((NSLOT,)),
            pltpu.SemaphoreType.DMA((NSLOT,)),
        ],
        compiler_params=pltpu.CompilerParams(needs_layout_passes=False,
                                             use_tc_tiling_on_sc=True),
    )(embt, tail)

    partials = pl.kernel(
        _loss_body,
        out_type=jax.ShapeDtypeStruct((NW, LANES), jnp.float32),
        mesh=mesh,
        scratch_types=[
            pltpu.VMEM((B_PER_W,), jnp.int32),
            pltpu.VMEM((B_PER_W,), jnp.int32),
            pltpu.VMEM((NEG_IDX_ROWS_W, 128), jnp.int32),
            pltpu.VMEM((NBUF, CHUNK, DIM), jnp.float32),
            pltpu.VMEM((NBUF, CHUNK, DIM), jnp.float32),
            pltpu.VMEM((NBUF, NEG_PER_CHUNK, DIM), jnp.float32),
            pltpu.VMEM((LANES,), jnp.float32),
            pltpu.SemaphoreType.DMA((NBUF,)),
        ],
        compiler_params=pltpu.CompilerParams(needs_layout_passes=False,
                                             use_tc_tiling_on_sc=False),
    )(table_flat.reshape(NUM_NODES, DIM), targets, contexts, negatives2d)
    return jnp.sum(partials) / (BATCH * NUM_NEG)


def kernel(embeddings, targets, contexts, negatives):
    targets = targets.astype(jnp.int32)
    contexts = contexts.astype(jnp.int32)
    negatives2d = negatives.astype(jnp.int32).reshape(-1, 128)
    return _margin_loss(embeddings, targets, contexts, negatives2d)


# speedup vs baseline: 2.1071x; 1.0167x over previous
import functools

import jax
import jax.numpy as jnp
from jax import lax
from jax.experimental import pallas as pl
from jax.experimental.pallas import tpu as pltpu
from jax.experimental.pallas import tpu_sc as plsc

NUM_NODES = 1000000
DIM = 64
BATCH = 16384
NUM_NEG = 20
MARGIN = 1.0

NC = 2
NS = 16
NW = NC * NS
LANES = 16

_COMPILER_PARAMS = pltpu.CompilerParams(needs_layout_passes=False,
                                        use_tc_tiling_on_sc=True)


NBLK = NUM_NODES // 128
MAIN_ITERS = NBLK // NW
EXTRA_BLKS = NBLK - MAIN_ITERS * NW
TAIL_NODES = NUM_NODES - NBLK * 128
BLK_WORDS = 128 * DIM
NSLOT = 4


def _transpose_body(embt_hbm, tail_hbm, out_hbm, in_v, out_v, isem, osem):
    wid = lax.axis_index("c") * NS + lax.axis_index("s")
    lane_iota = lax.iota(jnp.int32, LANES)

    def fire_in(b, s):
        pltpu.make_async_copy(
            embt_hbm.at[pl.ds(0, DIM), pl.ds(b * 128, 128)],
            in_v.at[s], isem.at[s]).start()

    def wait_in(s):
        pltpu.make_async_copy(
            embt_hbm.at[pl.ds(0, DIM), pl.ds(0, 128)],
            in_v.at[s], isem.at[s]).wait()

    def fire_out(b, s):
        pltpu.make_async_copy(
            out_v.at[pl.ds(s * BLK_WORDS, BLK_WORDS)],
            out_hbm.at[pl.ds(b * BLK_WORDS, BLK_WORDS)], osem.at[s]).start()

    def wait_out(s):
        pltpu.make_async_copy(
            out_v.at[pl.ds(0, BLK_WORDS)],
            out_hbm.at[pl.ds(0, BLK_WORDS)], osem.at[s]).wait()

    def transpose_block(s):
        src = in_v.at[s]
        obase = s * BLK_WORDS

        def j_body(j, _):
            for dd in range(4):
                dvec = (lane_iota + (j * 4 + dd)) & (DIM - 1)
                for nsub in range(8):
                    nvec = lane_iota + nsub * LANES
                    val = plsc.load_gather(src, [dvec, nvec])
                    plsc.store_scatter(out_v, [obase + nvec * DIM + dvec], val)
            return 0

        lax.fori_loop(0, DIM // 4, j_body, 0)

    for p in range(NSLOT - 1):
        fire_in(p * NW + wid, p)

    def main_body(i, _):
        s = lax.rem(i, NSLOT)

        @pl.when(i + NSLOT - 1 < MAIN_ITERS)
        def _():
            fire_in((i + NSLOT - 1) * NW + wid, lax.rem(i + NSLOT - 1, NSLOT))

        wait_in(s)

        @pl.when(i >= NSLOT)
        def _():
            wait_out(s)

        transpose_block(s)
        fire_out(i * NW + wid, s)
        return 0

    lax.fori_loop(0, MAIN_ITERS, main_body, 0)
    for s in range(NSLOT):
        wait_out(s)

    @pl.when(wid < EXTRA_BLKS)
    def _():
        b = MAIN_ITERS * NW + wid
        pltpu.sync_copy(embt_hbm.at[pl.ds(0, DIM), pl.ds(b * 128, 128)],
                        in_v.at[0])
        transpose_block(0)
        pltpu.sync_copy(out_v.at[pl.ds(0, BLK_WORDS)],
                        out_hbm.at[pl.ds(b * BLK_WORDS, BLK_WORDS)])

    @pl.when(wid == EXTRA_BLKS)
    def _():
        pltpu.sync_copy(tail_hbm, out_v.at[pl.ds(0, TAIL_NODES * DIM)])
        pltpu.sync_copy(out_v.at[pl.ds(0, TAIL_NODES * DIM)],
                        out_hbm.at[pl.ds(NBLK * BLK_WORDS, TAIL_NODES * DIM)])


B_PER_W = BATCH // NW
CHUNK = 32
N_CHUNKS = B_PER_W // CHUNK
NEG_PER_CHUNK = CHUNK * NUM_NEG
NEG_IDX_ROWS = NEG_PER_CHUNK // 128
NEG_IDX_ROWS_W = B_PER_W * NUM_NEG // 128
GROUPS = CHUNK // LANES
UNROLL = 4
NBUF = 2


def _loss_body(emb_hbm, tgt_hbm, ctx_hbm, neg_hbm, out_hbm,
             t_idx, c_idx, n_idx, t_rows, c_rows, n_rows, acc_v, sem):
    wid = lax.axis_index("c") * NS + lax.axis_index("s")
    lane_iota = lax.iota(jnp.int32, LANES)
    zeros = jnp.zeros((LANES,), jnp.float32)

    pltpu.sync_copy(tgt_hbm.at[pl.ds(wid * B_PER_W, B_PER_W)], t_idx)
    pltpu.sync_copy(ctx_hbm.at[pl.ds(wid * B_PER_W, B_PER_W)], c_idx)
    pltpu.sync_copy(neg_hbm.at[pl.ds(wid * NEG_IDX_ROWS_W, NEG_IDX_ROWS_W)],
                    n_idx)

    def fire(ci, buf):
        cps = [pltpu.make_async_copy(
                   emb_hbm.at[t_idx.at[pl.ds(ci * CHUNK, CHUNK)]],
                   t_rows.at[buf], sem.at[buf]),
               pltpu.make_async_copy(
                   emb_hbm.at[c_idx.at[pl.ds(ci * CHUNK, CHUNK)]],
                   c_rows.at[buf], sem.at[buf])]
        cps += [pltpu.make_async_copy(emb_hbm.at[n_idx.at[ci * NEG_IDX_ROWS + j]],
                                      n_rows.at[buf].at[pl.ds(j * 128, 128)],
                                      sem.at[buf])
                for j in range(NEG_IDX_ROWS)]
        for cp in cps:
            cp.start()
        return cps

    def wait(ci, buf):
        for cp in fire_descr(buf):
            cp.wait()

    def fire_descr(buf):
        cps = [pltpu.make_async_copy(
                   emb_hbm.at[t_idx.at[pl.ds(0, CHUNK)]], t_rows.at[buf],
                   sem.at[buf]),
               pltpu.make_async_copy(
                   emb_hbm.at[c_idx.at[pl.ds(0, CHUNK)]], c_rows.at[buf],
                   sem.at[buf])]
        cps += [pltpu.make_async_copy(emb_hbm.at[n_idx.at[j]],
                                      n_rows.at[buf].at[pl.ds(j * 128, 128)],
                                      sem.at[buf])
                for j in range(NEG_IDX_ROWS)]
        return cps

    def compute(buf, loss_acc):
        t_r, c_r, n_r = t_rows.at[buf], c_rows.at[buf], n_rows.at[buf]

        def group_body(g, acc):
            e_lanes = lane_iota + g * LANES
            n_base = e_lanes * NUM_NEG

            def d_body(j, carry):
                pos, negs = carry
                for dd in range(UNROLL):
                    dcol = (lane_iota + (j * UNROLL + dd)) & (DIM - 1)
                    t_d = plsc.load_gather(t_r, [e_lanes, dcol])
                    c_d = plsc.load_gather(c_r, [e_lanes, dcol])
                    pos = pos + t_d * c_d
                    negs = tuple(
                        negs[k]
                        + t_d * plsc.load_gather(n_r, [n_base + k, dcol])
                        for k in range(NUM_NEG))
                return pos, negs

            pos, negs = lax.fori_loop(
                0, DIM // UNROLL, d_body, (zeros, (zeros,) * NUM_NEG))
            contrib = zeros
            for k in range(NUM_NEG):
                contrib = contrib + jnp.maximum(MARGIN - pos + negs[k], 0.0)
            return acc + contrib

        return lax.fori_loop(0, GROUPS, group_body, loss_acc)

    fire(0, 0)

    def chunk_body(ci, loss_acc):
        buf = lax.rem(ci, NBUF)
        nbuf = lax.rem(ci + 1, NBUF)

        @pl.when(ci + 1 < N_CHUNKS)
        def _():
            fire(ci + 1, nbuf)

        wait(ci, buf)
        return compute(buf, loss_acc)

    loss = lax.fori_loop(0, N_CHUNKS, chunk_body, zeros)
    acc_v[...] = loss
    pltpu.sync_copy(acc_v, out_hbm.at[wid])


@jax.jit
def _margin_loss(embeddings, targets, contexts, negatives2d):
    mesh = plsc.VectorSubcoreMesh(core_axis_name="c", subcore_axis_name="s",
                                  num_cores=NC, num_subcores=NS)

    embt = embeddings.T
    tail = embeddings[NBLK * 128:].reshape(-1)

    table_flat = pl.kernel(
        _transpose_body,
        out_type=jax.ShapeDtypeStruct((NUM_NODES * DIM,), jnp.float32),
        mesh=mesh,
        scratch_types=[
            pltpu.VMEM((NSLOT, DIM, 128), jnp.float32),
            pltpu.VMEM((NSLOT * BLK_WORDS,), jnp.float32),
            pltpu.SemaphoreType.DMA((NSLOT,)),
            pltpu.SemaphoreType.DMA((NSLOT,)),
        ],
        compiler_params=pltpu.CompilerParams(needs_layout_passes=False,
                                             use_tc_tiling_on_sc=True),
    )(embt, tail)

    partials = pl.kernel(
        _loss_body,
        out_type=jax.ShapeDtypeStruct((NW, LANES), jnp.float32),
        mesh=mesh,
        scratch_types=[
            pltpu.VMEM((B_PER_W,), jnp.int32),
            pltpu.VMEM((B_PER_W,), jnp.int32),
            pltpu.VMEM((NEG_IDX_ROWS_W, 128), jnp.int32),
            pltpu.VMEM((NBUF, CHUNK, DIM), jnp.float32),
            pltpu.VMEM((NBUF, CHUNK, DIM), jnp.float32),
            pltpu.VMEM((NBUF, NEG_PER_CHUNK, DIM), jnp.float32),
            pltpu.VMEM((LANES,), jnp.float32),
            pltpu.SemaphoreType.DMA((NBUF,)),
        ],
        compiler_params=pltpu.CompilerParams(needs_layout_passes=False,
                                             use_tc_tiling_on_sc=False),
    )(table_flat.reshape(NUM_NODES, DIM), targets, contexts, negatives2d)
    return jnp.sum(partials) / (BATCH * NUM_NEG)


def kernel(embeddings, targets, contexts, negatives):
    targets = targets.astype(jnp.int32)
    contexts = contexts.astype(jnp.int32)
    negatives2d = negatives.astype(jnp.int32).reshape(-1, 128)
    return _margin_loss(embeddings, targets, contexts, negatives2d)


# speedup vs baseline: 2.1078x; 1.0003x over previous
import jax
import jax.numpy as jnp
from jax import lax
from jax.experimental import pallas as pl
from jax.experimental.pallas import tpu as pltpu
from jax.experimental.pallas import tpu_sc as plsc

NUM_NODES = 1000000
DIM = 64
BATCH = 16384
NUM_NEG = 20
MARGIN = 1.0

NC = 2
NS = 16
NW = NC * NS
LANES = 16

_COMPILER_PARAMS = pltpu.CompilerParams(needs_layout_passes=False,
                                        use_tc_tiling_on_sc=True)


NBLK = NUM_NODES // 128
MAIN_ITERS = NBLK // NW
EXTRA_BLKS = NBLK - MAIN_ITERS * NW
TAIL_NODES = NUM_NODES - NBLK * 128
BLK_WORDS = 128 * DIM
NSLOT = 4


def _transpose_body(embt_hbm, tail_hbm, out_hbm, in_v, out_v, isem, osem):
    wid = lax.axis_index("c") * NS + lax.axis_index("s")
    lane_iota = lax.iota(jnp.int32, LANES)

    def fire_in(b, s):
        pltpu.make_async_copy(
            embt_hbm.at[pl.ds(0, DIM), pl.ds(b * 128, 128)],
            in_v.at[s], isem.at[s]).start()

    def wait_in(s):
        pltpu.make_async_copy(
            embt_hbm.at[pl.ds(0, DIM), pl.ds(0, 128)],
            in_v.at[s], isem.at[s]).wait()

    def fire_out(b, s):
        pltpu.make_async_copy(
            out_v.at[pl.ds(s * BLK_WORDS, BLK_WORDS)],
            out_hbm.at[pl.ds(b * BLK_WORDS, BLK_WORDS)], osem.at[s]).start()

    def wait_out(s):
        pltpu.make_async_copy(
            out_v.at[pl.ds(0, BLK_WORDS)],
            out_hbm.at[pl.ds(0, BLK_WORDS)], osem.at[s]).wait()

    def transpose_block(s):
        src = in_v.at[s]
        obase = s * BLK_WORDS

        def j_body(j, _):
            for dd in range(4):
                dvec = (lane_iota + (j * 4 + dd)) & (DIM - 1)
                for nsub in range(8):
                    nvec = lane_iota + nsub * LANES
                    val = plsc.load_gather(src, [dvec, nvec])
                    plsc.store_scatter(out_v, [obase + nvec * DIM + dvec], val)
            return 0

        lax.fori_loop(0, DIM // 4, j_body, 0)

    for p in range(NSLOT - 1):
        fire_in(p * NW + wid, p)

    def main_body(i, _):
        s = lax.rem(i, NSLOT)

        @pl.when(i + NSLOT - 1 < MAIN_ITERS)
        def _():
            fire_in((i + NSLOT - 1) * NW + wid, lax.rem(i + NSLOT - 1, NSLOT))

        wait_in(s)

        @pl.when(i >= NSLOT)
        def _():
            wait_out(s)

        transpose_block(s)
        fire_out(i * NW + wid, s)
        return 0

    lax.fori_loop(0, MAIN_ITERS, main_body, 0)
    for s in range(NSLOT):
        wait_out(s)

    @pl.when(wid < EXTRA_BLKS)
    def _():
        b = MAIN_ITERS * NW + wid
        pltpu.sync_copy(embt_hbm.at[pl.ds(0, DIM), pl.ds(b * 128, 128)],
                        in_v.at[0])
        transpose_block(0)
        pltpu.sync_copy(out_v.at[pl.ds(0, BLK_WORDS)],
                        out_hbm.at[pl.ds(b * BLK_WORDS, BLK_WORDS)])

    @pl.when(wid == EXTRA_BLKS)
    def _():
        pltpu.sync_copy(tail_hbm, out_v.at[pl.ds(0, TAIL_NODES * DIM)])
        pltpu.sync_copy(out_v.at[pl.ds(0, TAIL_NODES * DIM)],
                        out_hbm.at[pl.ds(NBLK * BLK_WORDS, TAIL_NODES * DIM)])


B_PER_W = BATCH // NW
CHUNK = 32
N_CHUNKS = B_PER_W // CHUNK
NEG_PER_CHUNK = CHUNK * NUM_NEG
NEG_IDX_ROWS = NEG_PER_CHUNK // 128
NEG_IDX_ROWS_W = B_PER_W * NUM_NEG // 128
GROUPS = CHUNK // LANES
UNROLL = 4
NBUF = 2


def _loss_body(emb_hbm, tgt_hbm, ctx_hbm, neg_hbm, out_hbm,
             t_idx, c_idx, n_idx, t_rows, c_rows, n_rows, acc_v, sem):
    wid = lax.axis_index("c") * NS + lax.axis_index("s")
    lane_iota = lax.iota(jnp.int32, LANES)
    zeros = jnp.zeros((LANES,), jnp.float32)

    pltpu.sync_copy(tgt_hbm.at[pl.ds(wid * B_PER_W, B_PER_W)], t_idx)
    pltpu.sync_copy(ctx_hbm.at[pl.ds(wid * B_PER_W, B_PER_W)], c_idx)
    pltpu.sync_copy(neg_hbm.at[pl.ds(wid * NEG_IDX_ROWS_W, NEG_IDX_ROWS_W)],
                    n_idx)

    def fire(ci, buf):
        cps = [pltpu.make_async_copy(
                   emb_hbm.at[t_idx.at[pl.ds(ci * CHUNK, CHUNK)]],
                   t_rows.at[buf], sem.at[buf]),
               pltpu.make_async_copy(
                   emb_hbm.at[c_idx.at[pl.ds(ci * CHUNK, CHUNK)]],
                   c_rows.at[buf], sem.at[buf])]
        cps += [pltpu.make_async_copy(emb_hbm.at[n_idx.at[ci * NEG_IDX_ROWS + j]],
                                      n_rows.at[buf].at[pl.ds(j * 128, 128)],
                                      sem.at[buf])
                for j in range(NEG_IDX_ROWS)]
        for cp in cps:
            cp.start()
        return cps

    def wait(ci, buf):
        for cp in fire_descr(buf):
            cp.wait()

    def fire_descr(buf):
        cps = [pltpu.make_async_copy(
                   emb_hbm.at[t_idx.at[pl.ds(0, CHUNK)]], t_rows.at[buf],
                   sem.at[buf]),
               pltpu.make_async_copy(
                   emb_hbm.at[c_idx.at[pl.ds(0, CHUNK)]], c_rows.at[buf],
                   sem.at[buf])]
        cps += [pltpu.make_async_copy(emb_hbm.at[n_idx.at[j]],
                                      n_rows.at[buf].at[pl.ds(j * 128, 128)],
                                      sem.at[buf])
                for j in range(NEG_IDX_ROWS)]
        return cps

    def compute(buf, loss_acc):
        t_r, c_r, n_r = t_rows.at[buf], c_rows.at[buf], n_rows.at[buf]

        def group_body(g, acc):
            e_lanes = lane_iota + g * LANES
            n_base = e_lanes * NUM_NEG

            def d_body(j, carry):
                pos, negs = carry
                for dd in range(UNROLL):
                    dcol = (lane_iota + (j * UNROLL + dd)) & (DIM - 1)
                    t_d = plsc.load_gather(t_r, [e_lanes, dcol])
                    c_d = plsc.load_gather(c_r, [e_lanes, dcol])
                    pos = pos + t_d * c_d
                    negs = tuple(
                        negs[k]
                        + t_d * plsc.load_gather(n_r, [n_base + k, dcol])
                        for k in range(NUM_NEG))
                return pos, negs

            pos, negs = lax.fori_loop(
                0, DIM // UNROLL, d_body, (zeros, (zeros,) * NUM_NEG))
            contrib = zeros
            for k in range(NUM_NEG):
                contrib = contrib + jnp.maximum(MARGIN - pos + negs[k], 0.0)
            return acc + contrib

        return lax.fori_loop(0, GROUPS, group_body, loss_acc)

    fire(0, 0)

    def chunk_body(ci, loss_acc):
        buf = lax.rem(ci, NBUF)
        nbuf = lax.rem(ci + 1, NBUF)

        @pl.when(ci + 1 < N_CHUNKS)
        def _():
            fire(ci + 1, nbuf)

        wait(ci, buf)
        return compute(buf, loss_acc)

    loss = lax.fori_loop(0, N_CHUNKS, chunk_body, zeros)
    acc_v[...] = loss
    pltpu.sync_copy(acc_v, out_hbm.at[wid])


@jax.jit
def _margin_loss(embeddings, targets, contexts, negatives2d):
    mesh = plsc.VectorSubcoreMesh(core_axis_name="c", subcore_axis_name="s",
                                  num_cores=NC, num_subcores=NS)

    embt = embeddings.T
    tail = embeddings[NBLK * 128:].reshape(-1)

    table_flat = pl.kernel(
        _transpose_body,
        out_type=jax.ShapeDtypeStruct((NUM_NODES * DIM,), jnp.float32),
        mesh=mesh,
        scratch_types=[
            pltpu.VMEM((NSLOT, DIM, 128), jnp.float32),
            pltpu.VMEM((NSLOT * BLK_WORDS,), jnp.float32),
            pltpu.SemaphoreType.DMA((NSLOT,)),
            pltpu.SemaphoreType.DMA((NSLOT,)),
        ],
        compiler_params=pltpu.CompilerParams(needs_layout_passes=False,
                                             use_tc_tiling_on_sc=True),
    )(embt, tail)

    partials = pl.kernel(
        _loss_body,
        out_type=jax.ShapeDtypeStruct((NW, LANES), jnp.float32),
        mesh=mesh,
        scratch_types=[
            pltpu.VMEM((B_PER_W,), jnp.int32),
            pltpu.VMEM((B_PER_W,), jnp.int32),
            pltpu.VMEM((NEG_IDX_ROWS_W, 128), jnp.int32),
            pltpu.VMEM((NBUF, CHUNK, DIM), jnp.float32),
            pltpu.VMEM((NBUF, CHUNK, DIM), jnp.float32),
            pltpu.VMEM((NBUF, NEG_PER_CHUNK, DIM), jnp.float32),
            pltpu.VMEM((LANES,), jnp.float32),
            pltpu.SemaphoreType.DMA((NBUF,)),
        ],
        compiler_params=pltpu.CompilerParams(needs_layout_passes=False,
                                             use_tc_tiling_on_sc=False),
    )(table_flat.reshape(NUM_NODES, DIM), targets, contexts, negatives2d)
    return jnp.sum(partials) / (BATCH * NUM_NEG)


def kernel(embeddings, targets, contexts, negatives):
    targets = targets.astype(jnp.int32)
    contexts = contexts.astype(jnp.int32)
    negatives2d = negatives.astype(jnp.int32).reshape(-1, 128)
    return _margin_loss(embeddings, targets, contexts, negatives2d)
